# baseline (speedup 1.0000x reference)
import functools

import jax
import jax.numpy as jnp
from jax.experimental import pallas as pl
from jax.experimental.pallas import tpu as pltpu


_HB = 8


def _adapter_kernel(x_ref, w1k_ref, w2k_ref, o_ref, *, res_scale, hb):
    _, c, _, w = x_ref.shape
    x = x_ref[0].reshape(c * hb, w)
    xb = x.astype(jnp.bfloat16)
    z = jnp.dot(w1k_ref[...], xb, preferred_element_type=jnp.float32)
    zb = jnp.maximum(z, 0.0).astype(jnp.bfloat16)
    y = jnp.dot(w2k_ref[...], zb, preferred_element_type=jnp.float32)
    y = jnp.maximum(y, 0.0)
    o_ref[0] = (y + res_scale * x).reshape(c, hb, w).astype(o_ref.dtype)


def _adapt_one(feat, w1, w2, ratio, *, hb=_HB):
    bs, c, H, W = feat.shape
    assert H % hb == 0, (H, hb)
    eye = jnp.eye(hb, dtype=jnp.float32)
    w1k = jnp.kron(w1.astype(jnp.float32), eye).astype(jnp.bfloat16)
    w2k = jnp.kron(w2.astype(jnp.float32) * jnp.float32(ratio),
                   eye).astype(jnp.bfloat16)

    return pl.pallas_call(
        functools.partial(_adapter_kernel, res_scale=1.0 - float(ratio),
                          hb=hb),
        out_shape=jax.ShapeDtypeStruct((bs, c, H, W), feat.dtype),
        grid=(bs, H // hb),
        in_specs=[
            pl.BlockSpec((1, c, hb, W), lambda b, j: (b, 0, j, 0)),
            pl.BlockSpec(w1k.shape, lambda b, j: (0, 0)),
            pl.BlockSpec(w2k.shape, lambda b, j: (0, 0)),
        ],
        out_specs=pl.BlockSpec((1, c, hb, W), lambda b, j: (b, 0, j, 0)),
        compiler_params=pltpu.CompilerParams(
            dimension_semantics=("parallel", "parallel"),
        ),
    )(feat, w1k, w2k)


def kernel(src_p3_camera, src_p3_lidar, src_p4_camera, src_p4_lidar,
           src_p5_camera, src_p5_lidar,
           w1_p3_camera, w2_p3_camera, w1_p3_lidar, w2_p3_lidar,
           w1_p4_camera, w2_p4_camera, w1_p4_lidar, w2_p4_lidar):
    r_cam, r_lid = 0.2, 0.6
    return {
        "p3": {
            "camera": _adapt_one(src_p3_camera, w1_p3_camera, w2_p3_camera,
                                 r_cam),
            "lidar": _adapt_one(src_p3_lidar, w1_p3_lidar, w2_p3_lidar, r_lid),
        },
        "p4": {
            "camera": _adapt_one(src_p4_camera, w1_p4_camera, w2_p4_camera,
                                 r_cam),
            "lidar": _adapt_one(src_p4_lidar, w1_p4_lidar, w2_p4_lidar, r_lid),
        },
        "p5": {"camera": src_p5_camera, "lidar": src_p5_lidar},
    }


# speedup vs baseline: 1.3847x; 1.3847x over previous
import functools

import jax
import jax.numpy as jnp
from jax.experimental import pallas as pl
from jax.experimental.pallas import tpu as pltpu


_KB = 8


def _adapter_kernel(x_ref, w1k_ref, w2k_ref, o_ref, *, res_scale, hb):
    _, c, _, w = x_ref.shape
    w1k = w1k_ref[...]
    w2k = w2k_ref[...]
    for k in range(hb // _KB):
        x = x_ref[0, :, k * _KB:(k + 1) * _KB, :].reshape(c * _KB, w)
        xb = x.astype(jnp.bfloat16)
        z = jnp.dot(w1k, xb, preferred_element_type=jnp.float32)
        zb = jnp.maximum(z, 0.0).astype(jnp.bfloat16)
        y = jnp.dot(w2k, zb, preferred_element_type=jnp.float32)
        y = jnp.maximum(y, 0.0)
        o_ref[0, :, k * _KB:(k + 1) * _KB, :] = (
            (y + res_scale * x).reshape(c, _KB, w).astype(o_ref.dtype))


def _adapt_one(feat, w1, w2, ratio, *, hb=32):
    bs, c, H, W = feat.shape
    hb = min(hb, H)
    assert H % hb == 0 and hb % _KB == 0, (H, hb)
    eye = jnp.eye(_KB, dtype=jnp.float32)
    w1k = jnp.kron(w1.astype(jnp.float32), eye).astype(jnp.bfloat16)
    w2k = jnp.kron(w2.astype(jnp.float32) * jnp.float32(ratio),
                   eye).astype(jnp.bfloat16)

    return pl.pallas_call(
        functools.partial(_adapter_kernel, res_scale=1.0 - float(ratio),
                          hb=hb),
        out_shape=jax.ShapeDtypeStruct((bs, c, H, W), feat.dtype),
        grid=(bs, H // hb),
        in_specs=[
            pl.BlockSpec((1, c, hb, W), lambda b, j: (b, 0, j, 0)),
            pl.BlockSpec(w1k.shape, lambda b, j: (0, 0)),
            pl.BlockSpec(w2k.shape, lambda b, j: (0, 0)),
        ],
        out_specs=pl.BlockSpec((1, c, hb, W), lambda b, j: (b, 0, j, 0)),
        compiler_params=pltpu.CompilerParams(
            dimension_semantics=("parallel", "parallel"),
        ),
    )(feat, w1k, w2k)


def kernel(src_p3_camera, src_p3_lidar, src_p4_camera, src_p4_lidar,
           src_p5_camera, src_p5_lidar,
           w1_p3_camera, w2_p3_camera, w1_p3_lidar, w2_p3_lidar,
           w1_p4_camera, w2_p4_camera, w1_p4_lidar, w2_p4_lidar):
    r_cam, r_lid = 0.2, 0.6
    return {
        "p3": {
            "camera": _adapt_one(src_p3_camera, w1_p3_camera, w2_p3_camera,
                                 r_cam),
            "lidar": _adapt_one(src_p3_lidar, w1_p3_lidar, w2_p3_lidar, r_lid),
        },
        "p4": {
            "camera": _adapt_one(src_p4_camera, w1_p4_camera, w2_p4_camera,
                                 r_cam),
            "lidar": _adapt_one(src_p4_lidar, w1_p4_lidar, w2_p4_lidar, r_lid),
        },
        "p5": {"camera": src_p5_camera, "lidar": src_p5_lidar},
    }


# speedup vs baseline: 2.2916x; 1.6549x over previous
import functools

import jax
import jax.numpy as jnp
from jax.experimental import pallas as pl
from jax.experimental.pallas import tpu as pltpu


_KB = 8


def _adapter_kernel(x_ref, w1k_ref, w2k_ref, o_ref, *, res_scale, hb):
    _, c, _, w = x_ref.shape
    w1k = w1k_ref[...]
    w2k = w2k_ref[...]
    for k in range(hb // _KB):
        x = x_ref[0, :, k * _KB:(k + 1) * _KB, :].reshape(c * _KB, w)
        xb = x.astype(jnp.bfloat16)
        z = jnp.dot(w1k, xb, preferred_element_type=jnp.float32)
        zb = jnp.maximum(z, 0.0).astype(jnp.bfloat16)
        y = jnp.dot(w2k, zb, preferred_element_type=jnp.float32)
        y = jnp.maximum(y, 0.0)
        o_ref[0, :, k * _KB:(k + 1) * _KB, :] = (
            (y + res_scale * x).reshape(c, _KB, w).astype(o_ref.dtype))


def _kron_eye(w, kb):
    m, n = w.shape
    wf = w.astype(jnp.float32)
    wr = jnp.repeat(wf, kb, axis=0)
    src = jax.lax.broadcasted_iota(jnp.int32, (n, n * kb), 0)
    dst = jax.lax.broadcasted_iota(jnp.int32, (n, n * kb), 1)
    expand = (src == dst // kb).astype(jnp.float32)
    wk = jnp.dot(wr, expand)
    ri = jax.lax.broadcasted_iota(jnp.int32, (m * kb, n * kb), 0)
    ci = jax.lax.broadcasted_iota(jnp.int32, (m * kb, n * kb), 1)
    return jnp.where(ri % kb == ci % kb, wk, 0.0).astype(jnp.bfloat16)


def _adapt_one(feat, w1, w2, ratio, *, hb=32):
    bs, c, H, W = feat.shape
    out_shape = feat.shape
    if W < 128 and (H * W) % 128 == 0:
        H, W = (H * W) // 128, 128
        feat = feat.reshape(bs, c, H, W)
    hb = min(hb, H)
    assert H % hb == 0 and hb % _KB == 0, (H, hb)
    w1k = _kron_eye(w1, _KB)
    w2k = _kron_eye(w2.astype(jnp.float32) * jnp.float32(ratio), _KB)

    out = pl.pallas_call(
        functools.partial(_adapter_kernel, res_scale=1.0 - float(ratio),
                          hb=hb),
        out_shape=jax.ShapeDtypeStruct((bs, c, H, W), feat.dtype),
        grid=(bs, H // hb),
        in_specs=[
            pl.BlockSpec((1, c, hb, W), lambda b, j: (b, 0, j, 0)),
            pl.BlockSpec(w1k.shape, lambda b, j: (0, 0)),
            pl.BlockSpec(w2k.shape, lambda b, j: (0, 0)),
        ],
        out_specs=pl.BlockSpec((1, c, hb, W), lambda b, j: (b, 0, j, 0)),
        compiler_params=pltpu.CompilerParams(
            dimension_semantics=("parallel", "parallel"),
        ),
    )(feat, w1k, w2k)
    return out.reshape(out_shape)


def kernel(src_p3_camera, src_p3_lidar, src_p4_camera, src_p4_lidar,
           src_p5_camera, src_p5_lidar,
           w1_p3_camera, w2_p3_camera, w1_p3_lidar, w2_p3_lidar,
           w1_p4_camera, w2_p4_camera, w1_p4_lidar, w2_p4_lidar):
    r_cam, r_lid = 0.2, 0.6
    return {
        "p3": {
            "camera": _adapt_one(src_p3_camera, w1_p3_camera, w2_p3_camera,
                                 r_cam),
            "lidar": _adapt_one(src_p3_lidar, w1_p3_lidar, w2_p3_lidar, r_lid),
        },
        "p4": {
            "camera": _adapt_one(src_p4_camera, w1_p4_camera, w2_p4_camera,
                                 r_cam),
            "lidar": _adapt_one(src_p4_lidar, w1_p4_lidar, w2_p4_lidar, r_lid),
        },
        "p5": {"camera": src_p5_camera, "lidar": src_p5_lidar},
    }


# speedup vs baseline: 3.1093x; 1.3568x over previous
import functools

import jax
import jax.numpy as jnp
from jax.experimental import pallas as pl
from jax.experimental.pallas import tpu as pltpu


_KB = 8


def _adapter_kernel(x_ref, w1k_ref, w2k_ref, o_ref, *, res_scale, hb):
    _, c, _, w = x_ref.shape
    w1k = w1k_ref[...]
    w2k = w2k_ref[...]
    for k in range(hb // _KB):
        x = x_ref[0, :, k * _KB:(k + 1) * _KB, :].reshape(c * _KB, w)
        xb = x.astype(jnp.bfloat16)
        z = jnp.dot(w1k, xb, preferred_element_type=jnp.float32)
        zb = jnp.maximum(z, 0.0).astype(jnp.bfloat16)
        y = jnp.dot(w2k, zb, preferred_element_type=jnp.float32)
        y = jnp.maximum(y, 0.0)
        o_ref[0, :, k * _KB:(k + 1) * _KB, :] = (
            (y + res_scale * x).reshape(c, _KB, w).astype(o_ref.dtype))


def _kron_eye(w, kb):
    m, n = w.shape
    wf = w.astype(jnp.float32)
    wr = jnp.repeat(wf, kb, axis=0)
    src = jax.lax.broadcasted_iota(jnp.int32, (n, n * kb), 0)
    dst = jax.lax.broadcasted_iota(jnp.int32, (n, n * kb), 1)
    expand = (src == dst // kb).astype(jnp.float32)
    wk = jnp.dot(wr, expand)
    ri = jax.lax.broadcasted_iota(jnp.int32, (m * kb, n * kb), 0)
    ci = jax.lax.broadcasted_iota(jnp.int32, (m * kb, n * kb), 1)
    return jnp.where(ri % kb == ci % kb, wk, 0.0).astype(jnp.bfloat16)


def _token_kernel(x_ref, w1t_ref, w2t_ref, o_ref, *, res_scale):
    x = x_ref[0]
    xb = x.astype(jnp.bfloat16)
    z = jnp.dot(xb, w1t_ref[...], preferred_element_type=jnp.float32)
    zb = jnp.maximum(z, 0.0).astype(jnp.bfloat16)
    y = jnp.dot(zb, w2t_ref[...], preferred_element_type=jnp.float32)
    y = jnp.maximum(y, 0.0)
    o_ref[0] = (y + res_scale * x).astype(o_ref.dtype)


def _adapt_one_tok(feat, w1, w2, ratio, *, tb=2048):
    bs, c, H, W = feat.shape
    hw = H * W
    tb = min(tb, hw)
    assert hw % tb == 0, (hw, tb)
    xt = jnp.transpose(feat, (0, 2, 3, 1)).reshape(bs, hw, c)
    w1t = jnp.transpose(w1).astype(jnp.bfloat16)
    w2t = (jnp.transpose(w2).astype(jnp.float32)
           * jnp.float32(ratio)).astype(jnp.bfloat16)

    out = pl.pallas_call(
        functools.partial(_token_kernel, res_scale=1.0 - float(ratio)),
        out_shape=jax.ShapeDtypeStruct((bs, hw, c), feat.dtype),
        grid=(bs, hw // tb),
        in_specs=[
            pl.BlockSpec((1, tb, c), lambda b, j: (b, j, 0)),
            pl.BlockSpec(w1t.shape, lambda b, j: (0, 0)),
            pl.BlockSpec(w2t.shape, lambda b, j: (0, 0)),
        ],
        out_specs=pl.BlockSpec((1, tb, c), lambda b, j: (b, j, 0)),
        compiler_params=pltpu.CompilerParams(
            dimension_semantics=("parallel", "parallel"),
        ),
    )(xt, w1t, w2t)
    return jnp.transpose(out.reshape(bs, H, W, c), (0, 3, 1, 2))


def _adapt_one(feat, w1, w2, ratio, *, hb=32):
    bs, c, H, W = feat.shape
    out_shape = feat.shape
    if W != 128:
        return _adapt_one_tok(feat, w1, w2, ratio)
    hb = min(hb, H)
    assert H % hb == 0 and hb % _KB == 0, (H, hb)
    w1k = _kron_eye(w1, _KB)
    w2k = _kron_eye(w2.astype(jnp.float32) * jnp.float32(ratio), _KB)

    out = pl.pallas_call(
        functools.partial(_adapter_kernel, res_scale=1.0 - float(ratio),
                          hb=hb),
        out_shape=jax.ShapeDtypeStruct((bs, c, H, W), feat.dtype),
        grid=(bs, H // hb),
        in_specs=[
            pl.BlockSpec((1, c, hb, W), lambda b, j: (b, 0, j, 0)),
            pl.BlockSpec(w1k.shape, lambda b, j: (0, 0)),
            pl.BlockSpec(w2k.shape, lambda b, j: (0, 0)),
        ],
        out_specs=pl.BlockSpec((1, c, hb, W), lambda b, j: (b, 0, j, 0)),
        compiler_params=pltpu.CompilerParams(
            dimension_semantics=("parallel", "parallel"),
        ),
    )(feat, w1k, w2k)
    return out.reshape(out_shape)


def kernel(src_p3_camera, src_p3_lidar, src_p4_camera, src_p4_lidar,
           src_p5_camera, src_p5_lidar,
           w1_p3_camera, w2_p3_camera, w1_p3_lidar, w2_p3_lidar,
           w1_p4_camera, w2_p4_camera, w1_p4_lidar, w2_p4_lidar):
    r_cam, r_lid = 0.2, 0.6
    return {
        "p3": {
            "camera": _adapt_one(src_p3_camera, w1_p3_camera, w2_p3_camera,
                                 r_cam),
            "lidar": _adapt_one(src_p3_lidar, w1_p3_lidar, w2_p3_lidar, r_lid),
        },
        "p4": {
            "camera": _adapt_one(src_p4_camera, w1_p4_camera, w2_p4_camera,
                                 r_cam),
            "lidar": _adapt_one(src_p4_lidar, w1_p4_lidar, w2_p4_lidar, r_lid),
        },
        "p5": {"camera": src_p5_camera, "lidar": src_p5_lidar},
    }


# speedup vs baseline: 3.6008x; 1.1581x over previous
import functools

import jax
import jax.numpy as jnp
from jax.experimental import pallas as pl
from jax.experimental.pallas import tpu as pltpu


_KB = 8


def _adapter_kernel(x_ref, w1k_ref, w2k_ref, o_ref, *, res_scale, hb):
    _, c, _, w = x_ref.shape
    w1k = w1k_ref[...]
    w2k = w2k_ref[...]
    for k in range(hb // _KB):
        x = x_ref[0, :, k * _KB:(k + 1) * _KB, :].reshape(c * _KB, w)
        xb = x.astype(jnp.bfloat16)
        z = jnp.dot(w1k, xb, preferred_element_type=jnp.float32)
        zb = jnp.maximum(z, 0.0).astype(jnp.bfloat16)
        y = jnp.dot(w2k, zb, preferred_element_type=jnp.float32)
        y = jnp.maximum(y, 0.0)
        o_ref[0, :, k * _KB:(k + 1) * _KB, :] = (
            (y + res_scale * x).reshape(c, _KB, w).astype(o_ref.dtype))


def _kron_eye(w, kb):
    m, n = w.shape
    wf = w.astype(jnp.float32)
    wr = jnp.repeat(wf, kb, axis=0)
    src = jax.lax.broadcasted_iota(jnp.int32, (n, n * kb), 0)
    dst = jax.lax.broadcasted_iota(jnp.int32, (n, n * kb), 1)
    expand = (src == dst // kb).astype(jnp.float32)
    wk = jnp.dot(wr, expand)
    ri = jax.lax.broadcasted_iota(jnp.int32, (m * kb, n * kb), 0)
    ci = jax.lax.broadcasted_iota(jnp.int32, (m * kb, n * kb), 1)
    return jnp.where(ri % kb == ci % kb, wk, 0.0).astype(jnp.bfloat16)


def _token_kernel(x_ref, w1t_ref, w2t_ref, o_ref, *, res_scale):
    x = x_ref[0]
    xb = x.astype(jnp.bfloat16)
    z = jnp.dot(xb, w1t_ref[...], preferred_element_type=jnp.float32)
    zb = jnp.maximum(z, 0.0).astype(jnp.bfloat16)
    y = jnp.dot(zb, w2t_ref[...], preferred_element_type=jnp.float32)
    y = jnp.maximum(y, 0.0)
    o_ref[0] = (y + res_scale * x).astype(o_ref.dtype)


def _adapt_one_tok(feat, w1, w2, ratio, *, tb=2048):
    bs, c, H, W = feat.shape
    hw = H * W
    tb = min(tb, hw)
    assert hw % tb == 0, (hw, tb)
    xt = jnp.transpose(feat, (0, 2, 3, 1)).reshape(bs, hw, c)
    w1t = jnp.transpose(w1).astype(jnp.bfloat16)
    w2t = (jnp.transpose(w2).astype(jnp.float32)
           * jnp.float32(ratio)).astype(jnp.bfloat16)

    out = pl.pallas_call(
        functools.partial(_token_kernel, res_scale=1.0 - float(ratio)),
        out_shape=jax.ShapeDtypeStruct((bs, hw, c), feat.dtype),
        grid=(bs, hw // tb),
        in_specs=[
            pl.BlockSpec((1, tb, c), lambda b, j: (b, j, 0)),
            pl.BlockSpec(w1t.shape, lambda b, j: (0, 0)),
            pl.BlockSpec(w2t.shape, lambda b, j: (0, 0)),
        ],
        out_specs=pl.BlockSpec((1, tb, c), lambda b, j: (b, j, 0)),
        compiler_params=pltpu.CompilerParams(
            dimension_semantics=("parallel", "parallel"),
        ),
    )(xt, w1t, w2t)
    return jnp.transpose(out.reshape(bs, H, W, c), (0, 3, 1, 2))


def _adapt_one(feat, w1, w2, ratio, *, hb=64):
    bs, c, H, W = feat.shape
    out_shape = feat.shape
    if W != 128:
        return _adapt_one_tok(feat, w1, w2, ratio)
    hb = min(hb, H)
    assert H % hb == 0 and hb % _KB == 0, (H, hb)
    w1k = _kron_eye(w1, _KB)
    w2k = _kron_eye(w2.astype(jnp.float32) * jnp.float32(ratio), _KB)

    out = pl.pallas_call(
        functools.partial(_adapter_kernel, res_scale=1.0 - float(ratio),
                          hb=hb),
        out_shape=jax.ShapeDtypeStruct((bs, c, H, W), feat.dtype),
        grid=(bs, H // hb),
        in_specs=[
            pl.BlockSpec((1, c, hb, W), lambda b, j: (b, 0, j, 0)),
            pl.BlockSpec(w1k.shape, lambda b, j: (0, 0)),
            pl.BlockSpec(w2k.shape, lambda b, j: (0, 0)),
        ],
        out_specs=pl.BlockSpec((1, c, hb, W), lambda b, j: (b, 0, j, 0)),
        compiler_params=pltpu.CompilerParams(
            dimension_semantics=("parallel", "parallel"),
        ),
    )(feat, w1k, w2k)
    return out.reshape(out_shape)


def kernel(src_p3_camera, src_p3_lidar, src_p4_camera, src_p4_lidar,
           src_p5_camera, src_p5_lidar,
           w1_p3_camera, w2_p3_camera, w1_p3_lidar, w2_p3_lidar,
           w1_p4_camera, w2_p4_camera, w1_p4_lidar, w2_p4_lidar):
    r_cam, r_lid = 0.2, 0.6
    return {
        "p3": {
            "camera": _adapt_one(src_p3_camera, w1_p3_camera, w2_p3_camera,
                                 r_cam),
            "lidar": _adapt_one(src_p3_lidar, w1_p3_lidar, w2_p3_lidar, r_lid),
        },
        "p4": {
            "camera": _adapt_one(src_p4_camera, w1_p4_camera, w2_p4_camera,
                                 r_cam),
            "lidar": _adapt_one(src_p4_lidar, w1_p4_lidar, w2_p4_lidar, r_lid),
        },
        "p5": {"camera": src_p5_camera, "lidar": src_p5_lidar},
    }


# speedup vs baseline: 3.8024x; 1.0560x over previous
import functools

import jax
import jax.numpy as jnp
from jax.experimental import pallas as pl
from jax.experimental.pallas import tpu as pltpu


_KB = 8


def _adapter_kernel(x_ref, w1k_ref, w2k_ref, o_ref, *, res_scale, hb):
    _, c, _, w = x_ref.shape
    w1k = w1k_ref[...]
    w2k = w2k_ref[...]
    for k in range(hb // _KB):
        x = x_ref[0, :, k * _KB:(k + 1) * _KB, :].reshape(c * _KB, w)
        xb = x.astype(jnp.bfloat16)
        z = jnp.dot(w1k, xb, preferred_element_type=jnp.float32)
        zb = jnp.maximum(z, 0.0).astype(jnp.bfloat16)
        y = jnp.dot(w2k, zb, preferred_element_type=jnp.float32)
        y = jnp.maximum(y, 0.0)
        o_ref[0, :, k * _KB:(k + 1) * _KB, :] = (
            (y + res_scale * x).reshape(c, _KB, w).astype(o_ref.dtype))


def _kron_eye(w, kb):
    m, n = w.shape
    wf = w.astype(jnp.float32)
    wr = jnp.repeat(wf, kb, axis=0)
    src = jax.lax.broadcasted_iota(jnp.int32, (n, n * kb), 0)
    dst = jax.lax.broadcasted_iota(jnp.int32, (n, n * kb), 1)
    expand = (src == dst // kb).astype(jnp.float32)
    wk = jnp.dot(wr, expand)
    ri = jax.lax.broadcasted_iota(jnp.int32, (m * kb, n * kb), 0)
    ci = jax.lax.broadcasted_iota(jnp.int32, (m * kb, n * kb), 1)
    return jnp.where(ri % kb == ci % kb, wk, 0.0).astype(jnp.bfloat16)


def _token_kernel(x_ref, w1t_ref, w2t_ref, o_ref, *, res_scale):
    x = x_ref[0]
    xb = x.astype(jnp.bfloat16)
    z = jnp.dot(xb, w1t_ref[...], preferred_element_type=jnp.float32)
    zb = jnp.maximum(z, 0.0).astype(jnp.bfloat16)
    y = jnp.dot(zb, w2t_ref[...], preferred_element_type=jnp.float32)
    y = jnp.maximum(y, 0.0)
    o_ref[0] = (y + res_scale * x).astype(o_ref.dtype)


def _adapt_one_tok(feat, w1, w2, ratio, *, tb=2048):
    bs, c, H, W = feat.shape
    hw = H * W
    tb = min(tb, hw)
    assert hw % tb == 0, (hw, tb)
    xt = jnp.transpose(feat, (0, 2, 3, 1)).reshape(bs, hw, c)
    w1t = jnp.transpose(w1).astype(jnp.bfloat16)
    w2t = (jnp.transpose(w2).astype(jnp.float32)
           * jnp.float32(ratio)).astype(jnp.bfloat16)

    out = pl.pallas_call(
        functools.partial(_token_kernel, res_scale=1.0 - float(ratio)),
        out_shape=jax.ShapeDtypeStruct((bs, hw, c), feat.dtype),
        grid=(bs, hw // tb),
        in_specs=[
            pl.BlockSpec((1, tb, c), lambda b, j: (b, j, 0)),
            pl.BlockSpec(w1t.shape, lambda b, j: (0, 0)),
            pl.BlockSpec(w2t.shape, lambda b, j: (0, 0)),
        ],
        out_specs=pl.BlockSpec((1, tb, c), lambda b, j: (b, j, 0)),
        compiler_params=pltpu.CompilerParams(
            dimension_semantics=("parallel", "parallel"),
        ),
    )(xt, w1t, w2t)
    return jnp.transpose(out.reshape(bs, H, W, c), (0, 3, 1, 2))


def _adapt_one(feat, w1, w2, ratio, *, hb=128):
    bs, c, H, W = feat.shape
    out_shape = feat.shape
    if W != 128:
        return _adapt_one_tok(feat, w1, w2, ratio)
    hb = min(hb, H)
    assert H % hb == 0 and hb % _KB == 0, (H, hb)
    w1k = _kron_eye(w1, _KB)
    w2k = _kron_eye(w2.astype(jnp.float32) * jnp.float32(ratio), _KB)

    out = pl.pallas_call(
        functools.partial(_adapter_kernel, res_scale=1.0 - float(ratio),
                          hb=hb),
        out_shape=jax.ShapeDtypeStruct((bs, c, H, W), feat.dtype),
        grid=(bs, H // hb),
        in_specs=[
            pl.BlockSpec((1, c, hb, W), lambda b, j: (b, 0, j, 0)),
            pl.BlockSpec(w1k.shape, lambda b, j: (0, 0)),
            pl.BlockSpec(w2k.shape, lambda b, j: (0, 0)),
        ],
        out_specs=pl.BlockSpec((1, c, hb, W), lambda b, j: (b, 0, j, 0)),
        compiler_params=pltpu.CompilerParams(
            dimension_semantics=("parallel", "parallel"),
        ),
    )(feat, w1k, w2k)
    return out.reshape(out_shape)


def kernel(src_p3_camera, src_p3_lidar, src_p4_camera, src_p4_lidar,
           src_p5_camera, src_p5_lidar,
           w1_p3_camera, w2_p3_camera, w1_p3_lidar, w2_p3_lidar,
           w1_p4_camera, w2_p4_camera, w1_p4_lidar, w2_p4_lidar):
    r_cam, r_lid = 0.2, 0.6
    return {
        "p3": {
            "camera": _adapt_one(src_p3_camera, w1_p3_camera, w2_p3_camera,
                                 r_cam),
            "lidar": _adapt_one(src_p3_lidar, w1_p3_lidar, w2_p3_lidar, r_lid),
        },
        "p4": {
            "camera": _adapt_one(src_p4_camera, w1_p4_camera, w2_p4_camera,
                                 r_cam),
            "lidar": _adapt_one(src_p4_lidar, w1_p4_lidar, w2_p4_lidar, r_lid),
        },
        "p5": {"camera": src_p5_camera, "lidar": src_p5_lidar},
    }


# speedup vs baseline: 4.1000x; 1.0783x over previous
import functools

import jax
import jax.numpy as jnp
from jax.experimental import pallas as pl
from jax.experimental.pallas import tpu as pltpu


_KB = 8


def _adapter_kernel(x_ref, w1k_ref, w2k_ref, o_ref, *, res_scale, hb):
    _, c, _, w = x_ref.shape
    w1k = w1k_ref[...]
    w2k = w2k_ref[...]
    for k in range(hb // _KB):
        x = x_ref[0, :, k * _KB:(k + 1) * _KB, :].reshape(c * _KB, w)
        xb = x.astype(jnp.bfloat16)
        z = jnp.dot(w1k, xb, preferred_element_type=jnp.float32)
        zb = jnp.maximum(z, 0.0).astype(jnp.bfloat16)
        y = jnp.dot(w2k, zb, preferred_element_type=jnp.float32)
        y = jnp.maximum(y, 0.0)
        o_ref[0, :, k * _KB:(k + 1) * _KB, :] = (
            (y + res_scale * x).reshape(c, _KB, w).astype(o_ref.dtype))


def _kron_eye(w, kb):
    m, n = w.shape
    wf = w.astype(jnp.float32)
    wr = jnp.repeat(wf, kb, axis=0)
    src = jax.lax.broadcasted_iota(jnp.int32, (n, n * kb), 0)
    dst = jax.lax.broadcasted_iota(jnp.int32, (n, n * kb), 1)
    expand = (src == dst // kb).astype(jnp.float32)
    wk = jnp.dot(wr, expand)
    ri = jax.lax.broadcasted_iota(jnp.int32, (m * kb, n * kb), 0)
    ci = jax.lax.broadcasted_iota(jnp.int32, (m * kb, n * kb), 1)
    return jnp.where(ri % kb == ci % kb, wk, 0.0).astype(jnp.bfloat16)


def _token_kernel(x_ref, w1t_ref, w2t_ref, o_ref, *, res_scale):
    x = x_ref[0]
    xb = x.astype(jnp.bfloat16)
    z = jnp.dot(xb, w1t_ref[...], preferred_element_type=jnp.float32)
    zb = jnp.maximum(z, 0.0).astype(jnp.bfloat16)
    y = jnp.dot(zb, w2t_ref[...], preferred_element_type=jnp.float32)
    y = jnp.maximum(y, 0.0)
    o_ref[0] = (y + res_scale * x).astype(o_ref.dtype)


def _adapt_one_tok(feat, w1, w2, ratio, *, tb=4096):
    bs, c, H, W = feat.shape
    hw = H * W
    tb = min(tb, hw)
    assert hw % tb == 0, (hw, tb)
    xt = jnp.transpose(feat, (0, 2, 3, 1)).reshape(bs, hw, c)
    w1t = jnp.transpose(w1).astype(jnp.bfloat16)
    w2t = (jnp.transpose(w2).astype(jnp.float32)
           * jnp.float32(ratio)).astype(jnp.bfloat16)

    out = pl.pallas_call(
        functools.partial(_token_kernel, res_scale=1.0 - float(ratio)),
        out_shape=jax.ShapeDtypeStruct((bs, hw, c), feat.dtype),
        grid=(bs, hw // tb),
        in_specs=[
            pl.BlockSpec((1, tb, c), lambda b, j: (b, j, 0)),
            pl.BlockSpec(w1t.shape, lambda b, j: (0, 0)),
            pl.BlockSpec(w2t.shape, lambda b, j: (0, 0)),
        ],
        out_specs=pl.BlockSpec((1, tb, c), lambda b, j: (b, j, 0)),
        compiler_params=pltpu.CompilerParams(
            dimension_semantics=("parallel", "parallel"),
        ),
    )(xt, w1t, w2t)
    return jnp.transpose(out.reshape(bs, H, W, c), (0, 3, 1, 2))


def _adapt_one(feat, w1, w2, ratio, *, hb=128):
    bs, c, H, W = feat.shape
    out_shape = feat.shape
    if W != 128:
        return _adapt_one_tok(feat, w1, w2, ratio)
    hb = min(hb, H)
    assert H % hb == 0 and hb % _KB == 0, (H, hb)
    w1k = _kron_eye(w1, _KB)
    w2k = _kron_eye(w2.astype(jnp.float32) * jnp.float32(ratio), _KB)

    out = pl.pallas_call(
        functools.partial(_adapter_kernel, res_scale=1.0 - float(ratio),
                          hb=hb),
        out_shape=jax.ShapeDtypeStruct((bs, c, H, W), feat.dtype),
        grid=(bs, H // hb),
        in_specs=[
            pl.BlockSpec((1, c, hb, W), lambda b, j: (b, 0, j, 0)),
            pl.BlockSpec(w1k.shape, lambda b, j: (0, 0)),
            pl.BlockSpec(w2k.shape, lambda b, j: (0, 0)),
        ],
        out_specs=pl.BlockSpec((1, c, hb, W), lambda b, j: (b, 0, j, 0)),
        compiler_params=pltpu.CompilerParams(
            dimension_semantics=("parallel", "parallel"),
        ),
    )(feat, w1k, w2k)
    return out.reshape(out_shape)


def kernel(src_p3_camera, src_p3_lidar, src_p4_camera, src_p4_lidar,
           src_p5_camera, src_p5_lidar,
           w1_p3_camera, w2_p3_camera, w1_p3_lidar, w2_p3_lidar,
           w1_p4_camera, w2_p4_camera, w1_p4_lidar, w2_p4_lidar):
    r_cam, r_lid = 0.2, 0.6
    return {
        "p3": {
            "camera": _adapt_one(src_p3_camera, w1_p3_camera, w2_p3_camera,
                                 r_cam),
            "lidar": _adapt_one(src_p3_lidar, w1_p3_lidar, w2_p3_lidar, r_lid),
        },
        "p4": {
            "camera": _adapt_one(src_p4_camera, w1_p4_camera, w2_p4_camera,
                                 r_cam),
            "lidar": _adapt_one(src_p4_lidar, w1_p4_lidar, w2_p4_lidar, r_lid),
        },
        "p5": {"camera": src_p5_camera, "lidar": src_p5_lidar},
    }


# speedup vs baseline: 4.2960x; 1.0478x over previous
import functools

import jax
import jax.numpy as jnp
from jax.experimental import pallas as pl
from jax.experimental.pallas import tpu as pltpu


_KB = 8


def _adapter_kernel(x_ref, w1k_ref, w2k_ref, o_ref, *, res_scale, hb):
    _, c, _, w = x_ref.shape
    w1k = w1k_ref[...]
    w2k = w2k_ref[...]
    for g in range(hb // (2 * _KB)):
        k0, k1 = 2 * g, 2 * g + 1
        xa = x_ref[0, :, k0 * _KB:(k0 + 1) * _KB, :].reshape(c * _KB, w)
        xc = x_ref[0, :, k1 * _KB:(k1 + 1) * _KB, :].reshape(c * _KB, w)
        x2 = jnp.concatenate([xa, xc], axis=1)
        xb = x2.astype(jnp.bfloat16)
        z = jnp.dot(w1k, xb, preferred_element_type=jnp.float32)
        zb = jnp.maximum(z, 0.0).astype(jnp.bfloat16)
        y = jnp.dot(w2k, zb, preferred_element_type=jnp.float32)
        o = jnp.maximum(y, 0.0) + res_scale * x2
        o_ref[0, :, k0 * _KB:(k0 + 1) * _KB, :] = (
            o[:, :w].reshape(c, _KB, w).astype(o_ref.dtype))
        o_ref[0, :, k1 * _KB:(k1 + 1) * _KB, :] = (
            o[:, w:].reshape(c, _KB, w).astype(o_ref.dtype))


def _kron_eye(w, kb):
    m, n = w.shape
    wf = w.astype(jnp.float32)
    wr = jnp.repeat(wf, kb, axis=0)
    src = jax.lax.broadcasted_iota(jnp.int32, (n, n * kb), 0)
    dst = jax.lax.broadcasted_iota(jnp.int32, (n, n * kb), 1)
    expand = (src == dst // kb).astype(jnp.float32)
    wk = jnp.dot(wr, expand)
    ri = jax.lax.broadcasted_iota(jnp.int32, (m * kb, n * kb), 0)
    ci = jax.lax.broadcasted_iota(jnp.int32, (m * kb, n * kb), 1)
    return jnp.where(ri % kb == ci % kb, wk, 0.0).astype(jnp.bfloat16)


def _token_kernel(x_ref, w1t_ref, w2t_ref, o_ref, *, res_scale):
    x = x_ref[0]
    xb = x.astype(jnp.bfloat16)
    z = jnp.dot(xb, w1t_ref[...], preferred_element_type=jnp.float32)
    zb = jnp.maximum(z, 0.0).astype(jnp.bfloat16)
    y = jnp.dot(zb, w2t_ref[...], preferred_element_type=jnp.float32)
    y = jnp.maximum(y, 0.0)
    o_ref[0] = (y + res_scale * x).astype(o_ref.dtype)


def _adapt_one_tok(feat, w1, w2, ratio, *, tb=4096):
    bs, c, H, W = feat.shape
    hw = H * W
    tb = min(tb, hw)
    assert hw % tb == 0, (hw, tb)
    xt = jnp.transpose(feat, (0, 2, 3, 1)).reshape(bs, hw, c)
    w1t = jnp.transpose(w1).astype(jnp.bfloat16)
    w2t = (jnp.transpose(w2).astype(jnp.float32)
           * jnp.float32(ratio)).astype(jnp.bfloat16)

    out = pl.pallas_call(
        functools.partial(_token_kernel, res_scale=1.0 - float(ratio)),
        out_shape=jax.ShapeDtypeStruct((bs, hw, c), feat.dtype),
        grid=(bs, hw // tb),
        in_specs=[
            pl.BlockSpec((1, tb, c), lambda b, j: (b, j, 0)),
            pl.BlockSpec(w1t.shape, lambda b, j: (0, 0)),
            pl.BlockSpec(w2t.shape, lambda b, j: (0, 0)),
        ],
        out_specs=pl.BlockSpec((1, tb, c), lambda b, j: (b, j, 0)),
        compiler_params=pltpu.CompilerParams(
            dimension_semantics=("parallel", "parallel"),
        ),
    )(xt, w1t, w2t)
    return jnp.transpose(out.reshape(bs, H, W, c), (0, 3, 1, 2))


def _adapt_one(feat, w1, w2, ratio, *, hb=128):
    bs, c, H, W = feat.shape
    out_shape = feat.shape
    if W != 128:
        return _adapt_one_tok(feat, w1, w2, ratio)
    hb = min(hb, H)
    assert H % hb == 0 and hb % _KB == 0, (H, hb)
    w1k = _kron_eye(w1, _KB)
    w2k = _kron_eye(w2.astype(jnp.float32) * jnp.float32(ratio), _KB)

    out = pl.pallas_call(
        functools.partial(_adapter_kernel, res_scale=1.0 - float(ratio),
                          hb=hb),
        out_shape=jax.ShapeDtypeStruct((bs, c, H, W), feat.dtype),
        grid=(bs, H // hb),
        in_specs=[
            pl.BlockSpec((1, c, hb, W), lambda b, j: (b, 0, j, 0)),
            pl.BlockSpec(w1k.shape, lambda b, j: (0, 0)),
            pl.BlockSpec(w2k.shape, lambda b, j: (0, 0)),
        ],
        out_specs=pl.BlockSpec((1, c, hb, W), lambda b, j: (b, 0, j, 0)),
        compiler_params=pltpu.CompilerParams(
            dimension_semantics=("parallel", "parallel"),
        ),
    )(feat, w1k, w2k)
    return out.reshape(out_shape)


def kernel(src_p3_camera, src_p3_lidar, src_p4_camera, src_p4_lidar,
           src_p5_camera, src_p5_lidar,
           w1_p3_camera, w2_p3_camera, w1_p3_lidar, w2_p3_lidar,
           w1_p4_camera, w2_p4_camera, w1_p4_lidar, w2_p4_lidar):
    r_cam, r_lid = 0.2, 0.6
    return {
        "p3": {
            "camera": _adapt_one(src_p3_camera, w1_p3_camera, w2_p3_camera,
                                 r_cam),
            "lidar": _adapt_one(src_p3_lidar, w1_p3_lidar, w2_p3_lidar, r_lid),
        },
        "p4": {
            "camera": _adapt_one(src_p4_camera, w1_p4_camera, w2_p4_camera,
                                 r_cam),
            "lidar": _adapt_one(src_p4_lidar, w1_p4_lidar, w2_p4_lidar, r_lid),
        },
        "p5": {"camera": src_p5_camera, "lidar": src_p5_lidar},
    }
